# baseline (speedup 1.0000x reference)
import jax
import jax.numpy as jnp
from jax import lax
from jax.experimental import pallas as pl
from jax.experimental.pallas import tpu as pltpu
from jax.experimental.pallas import tpu_sc as plsc

_N = 10000
_NP = 10240
_E = 320000
_DIN = 128
_HID = 16
_OUT = 128

_TILES = 32
_CHUNK = 128
_SUP = 4
_NSUP = 20
_NSUPP = 22
_SCH = _SUP * _CHUNK
_NCHUNK = 80
_NCHUNKP = 88
_EPT = _CHUNK * _NCHUNK
_EP = _EPT * _TILES
_RPT = _NP // _TILES
_RPS = _NP // 16

_mesh = plsc.VectorSubcoreMesh(core_axis_name="c", subcore_axis_name="s")
_f32 = jnp.float32


def _hop_scatter_body(g_hbm, src_hbm, dst_hbm, zer_hbm, p_hbm,
                      src_v, dst_v, rb0, rb1, rb2, rb3, acc_sh,
                      sg0, sg1, sg2, sg3, ss0, ss1, ss2, ss3):
    c = lax.axis_index("c")
    s = lax.axis_index("s")
    wid = c * 16 + s
    pltpu.sync_copy(src_hbm.at[wid], src_v)
    pltpu.sync_copy(dst_hbm.at[wid], dst_v)
    r0 = s * _RPS
    pltpu.sync_copy(zer_hbm.at[pl.ds(r0, _RPS)], acc_sh.at[pl.ds(r0, _RPS)])
    plsc.subcore_barrier()

    rows = [rb0, rb1, rb2, rb3]
    sg = [sg0, sg1, sg2, sg3]
    ss = [ss0, ss1, ss2, ss3]

    def g_start(e, j):
        pltpu.async_copy(g_hbm.at[src_v.at[e]], rows[j], sg[j])

    def g_wait(e, j):
        pltpu.make_async_copy(g_hbm.at[src_v.at[e]], rows[j], sg[j]).wait()

    def s_start(e, j):
        pltpu.async_copy(rows[j], acc_sh.at[dst_v.at[e]], ss[j], add=True)

    def s_wait(e, j):
        pltpu.make_async_copy(rows[j], acc_sh.at[dst_v.at[e]], ss[j]).wait()

    g_start(0, 0)
    g_start(1, 1)
    g_wait(0, 0); s_start(0, 0); g_start(2, 2)
    g_wait(1, 1); s_start(1, 1); g_start(3, 3)
    g_wait(2, 2); s_start(2, 2); s_wait(0, 0); g_start(4, 0)
    g_wait(3, 3); s_start(3, 3); s_wait(1, 1); g_start(5, 1)

    def step(i, carry):
        e = 4 * i
        for j in range(4):
            g_wait(e + j, j)
            s_start(e + j, j)
            s_wait(e + j - 2, (j + 2) % 4)
            g_start(e + j + 2, (j + 2) % 4)
        return carry

    lax.fori_loop(1, _NSUP // 4, step, 0)
    s_wait(_NSUP - 2, 2)
    s_wait(_NSUP - 1, 3)
    g_wait(_NSUP, 0)
    g_wait(_NSUP + 1, 1)
    plsc.subcore_barrier()
    pltpu.sync_copy(acc_sh.at[pl.ds(r0, _RPS)], p_hbm.at[c, pl.ds(r0, _RPS)])


_scatter_call = pl.kernel(
    _hop_scatter_body,
    out_type=jax.ShapeDtypeStruct((2, _NP, _HID), _f32),
    mesh=_mesh,
    compiler_params=pltpu.CompilerParams(use_tc_tiling_on_sc=False),
    name="hop_scatter",
    scratch_types=[
        pltpu.VMEM((_NSUPP, _SCH), jnp.int32),
        pltpu.VMEM((_NSUPP, _SCH), jnp.int32),
        pltpu.VMEM((_SCH, _HID), _f32),
        pltpu.VMEM((_SCH, _HID), _f32),
        pltpu.VMEM((_SCH, _HID), _f32),
        pltpu.VMEM((_SCH, _HID), _f32),
        pltpu.VMEM_SHARED((_NP, _HID), _f32),
        pltpu.SemaphoreType.DMA,
        pltpu.SemaphoreType.DMA,
        pltpu.SemaphoreType.DMA,
        pltpu.SemaphoreType.DMA,
        pltpu.SemaphoreType.DMA,
        pltpu.SemaphoreType.DMA,
        pltpu.SemaphoreType.DMA,
        pltpu.SemaphoreType.DMA,
    ],
)


def _deg_scatter_body(ones_hbm, dst_hbm, zer_hbm, p_hbm,
                      dst_v, rows_v, acc_sh, sem):
    c = lax.axis_index("c")
    s = lax.axis_index("s")
    wid = c * 16 + s
    pltpu.sync_copy(dst_hbm.at[wid], dst_v)
    pltpu.sync_copy(ones_hbm, rows_v)
    r0 = s * _RPS
    pltpu.sync_copy(zer_hbm.at[pl.ds(r0, _RPS)], acc_sh.at[pl.ds(r0, _RPS)])
    plsc.subcore_barrier()

    def fire(j, carry):
        pltpu.async_copy(rows_v, acc_sh.at[dst_v.at[j]], sem, add=True)
        return carry

    def drain(j, carry):
        pltpu.make_async_copy(rows_v, acc_sh.at[dst_v.at[j]], sem).wait()
        return carry

    lax.fori_loop(0, _NSUP, fire, 0)
    lax.fori_loop(0, _NSUP, drain, 0)
    plsc.subcore_barrier()
    pltpu.sync_copy(acc_sh.at[pl.ds(r0, _RPS)], p_hbm.at[c, pl.ds(r0, _RPS)])


_deg_call = pl.kernel(
    _deg_scatter_body,
    out_type=jax.ShapeDtypeStruct((2, _NP, _HID), _f32),
    mesh=_mesh,
    compiler_params=pltpu.CompilerParams(use_tc_tiling_on_sc=False),
    name="deg_scatter",
    scratch_types=[
        pltpu.VMEM((_NSUPP, _SCH), jnp.int32),
        pltpu.VMEM((_SCH, _HID), _f32),
        pltpu.VMEM_SHARED((_NP, _HID), _f32),
        pltpu.SemaphoreType.DMA,
    ],
)


def _make_combine(has_z, relu):
    def body(p_hbm, z_hbm, d_hbm, t_hbm, g_hbm,
             p0_v, p1_v, z_v, d_v, t_v, g_v):
        c = lax.axis_index("c")
        s = lax.axis_index("s")
        wid = c * 16 + s
        r0 = wid * _RPT
        pltpu.sync_copy(p_hbm.at[0, pl.ds(r0, _RPT)], p0_v)
        pltpu.sync_copy(p_hbm.at[1, pl.ds(r0, _RPT)], p1_v)
        if has_z:
            pltpu.sync_copy(z_hbm.at[pl.ds(r0, _RPT)], z_v)
        pltpu.sync_copy(d_hbm.at[pl.ds(r0, _RPT)], d_v)

        def row(r, carry):
            acc = (p0_v[r, :] + p1_v[r, :]) * d_v[r, :]
            if has_z:
                acc = acc + z_v[r, :]
            if relu:
                acc = jnp.maximum(acc, 0.0)
            t_v[r, :] = acc
            g_v[r, :] = acc * d_v[r, :]
            return carry

        lax.fori_loop(0, _RPT, row, 0)
        pltpu.sync_copy(t_v, t_hbm.at[pl.ds(r0, _RPT)])
        pltpu.sync_copy(g_v, g_hbm.at[pl.ds(r0, _RPT)])

    return pl.kernel(
        body,
        out_type=(jax.ShapeDtypeStruct((_NP, _HID), _f32),
                  jax.ShapeDtypeStruct((_NP, _HID), _f32)),
        mesh=_mesh,
        compiler_params=pltpu.CompilerParams(use_tc_tiling_on_sc=False),
        name="combine" + ("_z" if has_z else "") + ("_relu" if relu else ""),
        scratch_types=[
            pltpu.VMEM((_RPT, _HID), _f32),
            pltpu.VMEM((_RPT, _HID), _f32),
            pltpu.VMEM((_RPT, _HID), _f32),
            pltpu.VMEM((_RPT, _HID), _f32),
            pltpu.VMEM((_RPT, _HID), _f32),
            pltpu.VMEM((_RPT, _HID), _f32),
        ],
    )


_combine_z = _make_combine(True, False)
_combine_z_relu = _make_combine(True, True)
_combine_noz = _make_combine(False, False)


def _tc_prep_body(x_ref, w_ref, degp_ref, zs_ref, g3_ref, d16_ref):
    w = w_ref[...]
    wr = w.reshape(4, _DIN, _HID).transpose(1, 0, 2).reshape(_DIN, 4 * _HID)
    xz = jnp.dot(x_ref[...], wr, preferred_element_type=_f32)
    deg = degp_ref[0] + degp_ref[1]
    dinv = jnp.where(deg > 0.0, lax.rsqrt(deg), 0.0)
    d16_ref[...] = dinv
    for k in range(4):
        zs_ref[k] = xz[:, _HID * k:_HID * (k + 1)]
    g3_ref[...] = dinv * xz[:, 3 * _HID:4 * _HID]


_prep_call = pl.pallas_call(
    _tc_prep_body,
    out_shape=(jax.ShapeDtypeStruct((4, _NP, _HID), _f32),
               jax.ShapeDtypeStruct((_NP, _HID), _f32),
               jax.ShapeDtypeStruct((_NP, _HID), _f32)),
)


def _tc_final_body(u0, u1, u2, u3, w_ref, out_ref):
    h = jnp.concatenate([u0[...], u1[...], u2[...], u3[...]], axis=1)
    out_ref[...] = jnp.dot(h, w_ref[...], preferred_element_type=_f32)


_final_call = pl.pallas_call(
    _tc_final_body,
    out_shape=jax.ShapeDtypeStruct((_NP, _OUT), _f32),
)


def kernel(x, edge_index, W1, W2):
    src = edge_index[0].astype(jnp.int32)
    dst = edge_index[1].astype(jnp.int32)
    pad = jnp.full((_EP - _E,), _NP - 1, jnp.int32)
    dummy = jnp.full((_TILES, _NCHUNKP - _NCHUNK, _CHUNK), _NP - 1, jnp.int32)
    src3 = jnp.concatenate(
        [jnp.concatenate([src, pad]).reshape(_TILES, _NCHUNK, _CHUNK), dummy],
        axis=1).reshape(_TILES, _NSUPP, _SCH)
    dst3 = jnp.concatenate(
        [jnp.concatenate([dst, pad]).reshape(_TILES, _NCHUNK, _CHUNK), dummy],
        axis=1).reshape(_TILES, _NSUPP, _SCH)
    xp = jnp.zeros((_NP, _DIN), _f32).at[:_N].set(x)
    zer = jnp.zeros((_NP, _HID), _f32)
    ones = jnp.ones((_SCH, _HID), _f32)

    degp = _deg_call(ones, dst3, zer)
    zs, g, d16 = _prep_call(xp, W1, degp)

    p = _scatter_call(g, src3, dst3, zer)
    _, g = _combine_z(p, zs[2], d16)
    p = _scatter_call(g, src3, dst3, zer)
    _, g = _combine_z(p, zs[1], d16)
    p = _scatter_call(g, src3, dst3, zer)
    u0, g = _combine_z_relu(p, zs[0], d16)

    us = [u0]
    for _ in range(3):
        p = _scatter_call(g, src3, dst3, zer)
        u, g = _combine_noz(p, zer, d16)
        us.append(u)

    out = _final_call(us[0], us[1], us[2], us[3], W2)
    return out[:_N]


# speedup vs baseline: 1.9291x; 1.9291x over previous
import jax
import jax.numpy as jnp
from jax import lax
from jax.experimental import pallas as pl
from jax.experimental.pallas import tpu as pltpu
from jax.experimental.pallas import tpu_sc as plsc

_N = 10000
_NP = 10240
_E = 320000
_DIN = 128
_HID = 16
_OUT = 128

_TILES = 32
_CHUNK = 128
_SUP = 1
_NSUP = 80
_NSUPP = 82
_SCH = _SUP * _CHUNK
_NCHUNK = 80
_NCHUNKP = 82
_EPT = _CHUNK * _NCHUNK
_EP = _EPT * _TILES
_RPT = _NP // _TILES
_RPS = _NP // 16

_mesh = plsc.VectorSubcoreMesh(core_axis_name="c", subcore_axis_name="s")
_f32 = jnp.float32


def _hop_scatter_body(g_hbm, src_hbm, dst_hbm, zer_hbm, p_hbm,
                      src_v, dst_v, rb0, rb1, rb2, rb3, acc_sh,
                      sg0, sg1, sg2, sg3, ss0, ss1, ss2, ss3):
    c = lax.axis_index("c")
    s = lax.axis_index("s")
    wid = c * 16 + s
    pltpu.sync_copy(src_hbm.at[wid], src_v)
    pltpu.sync_copy(dst_hbm.at[wid], dst_v)
    r0 = s * _RPS
    pltpu.sync_copy(zer_hbm.at[pl.ds(r0, _RPS)], acc_sh.at[pl.ds(r0, _RPS)])
    plsc.subcore_barrier()

    rows = [rb0, rb1, rb2, rb3]
    sg = [sg0, sg1, sg2, sg3]
    ss = [ss0, ss1, ss2, ss3]

    def g_start(e, j):
        pltpu.async_copy(g_hbm.at[src_v.at[e]], rows[j], sg[j])

    def g_wait(e, j):
        pltpu.make_async_copy(g_hbm.at[src_v.at[e]], rows[j], sg[j]).wait()

    def s_start(e, j):
        pltpu.async_copy(rows[j], acc_sh.at[dst_v.at[e]], ss[j], add=True)

    def s_wait(e, j):
        pltpu.make_async_copy(rows[j], acc_sh.at[dst_v.at[e]], ss[j]).wait()

    g_start(0, 0)
    g_start(1, 1)
    g_wait(0, 0); s_start(0, 0); g_start(2, 2)
    g_wait(1, 1); s_start(1, 1); g_start(3, 3)
    g_wait(2, 2); s_start(2, 2); s_wait(0, 0); g_start(4, 0)
    g_wait(3, 3); s_start(3, 3); s_wait(1, 1); g_start(5, 1)

    def step(i, carry):
        e = 4 * i
        for j in range(4):
            g_wait(e + j, j)
            s_start(e + j, j)
            s_wait(e + j - 2, (j + 2) % 4)
            g_start(e + j + 2, (j + 2) % 4)
        return carry

    lax.fori_loop(1, _NSUP // 4, step, 0)
    s_wait(_NSUP - 2, 2)
    s_wait(_NSUP - 1, 3)
    g_wait(_NSUP, 0)
    g_wait(_NSUP + 1, 1)
    plsc.subcore_barrier()
    pltpu.sync_copy(acc_sh.at[pl.ds(r0, _RPS)], p_hbm.at[c, pl.ds(r0, _RPS)])


_scatter_call = pl.kernel(
    _hop_scatter_body,
    out_type=jax.ShapeDtypeStruct((2, _NP, _HID), _f32),
    mesh=_mesh,
    compiler_params=pltpu.CompilerParams(use_tc_tiling_on_sc=False),
    name="hop_scatter",
    scratch_types=[
        pltpu.VMEM((_NSUPP, _SCH), jnp.int32),
        pltpu.VMEM((_NSUPP, _SCH), jnp.int32),
        pltpu.VMEM((_SCH, _HID), _f32),
        pltpu.VMEM((_SCH, _HID), _f32),
        pltpu.VMEM((_SCH, _HID), _f32),
        pltpu.VMEM((_SCH, _HID), _f32),
        pltpu.VMEM_SHARED((_NP, _HID), _f32),
        pltpu.SemaphoreType.DMA,
        pltpu.SemaphoreType.DMA,
        pltpu.SemaphoreType.DMA,
        pltpu.SemaphoreType.DMA,
        pltpu.SemaphoreType.DMA,
        pltpu.SemaphoreType.DMA,
        pltpu.SemaphoreType.DMA,
        pltpu.SemaphoreType.DMA,
    ],
)


def _deg_scatter_body(ones_hbm, dst_hbm, zer_hbm, p_hbm,
                      dst_v, rows_v, acc_sh, sem):
    c = lax.axis_index("c")
    s = lax.axis_index("s")
    wid = c * 16 + s
    pltpu.sync_copy(dst_hbm.at[wid], dst_v)
    pltpu.sync_copy(ones_hbm, rows_v)
    r0 = s * _RPS
    pltpu.sync_copy(zer_hbm.at[pl.ds(r0, _RPS)], acc_sh.at[pl.ds(r0, _RPS)])
    plsc.subcore_barrier()

    def fire(j, carry):
        pltpu.async_copy(rows_v, acc_sh.at[dst_v.at[j]], sem, add=True)
        return carry

    def drain(j, carry):
        pltpu.make_async_copy(rows_v, acc_sh.at[dst_v.at[j]], sem).wait()
        return carry

    lax.fori_loop(0, _NSUP, fire, 0)
    lax.fori_loop(0, _NSUP, drain, 0)
    plsc.subcore_barrier()
    pltpu.sync_copy(acc_sh.at[pl.ds(r0, _RPS)], p_hbm.at[c, pl.ds(r0, _RPS)])


_deg_call = pl.kernel(
    _deg_scatter_body,
    out_type=jax.ShapeDtypeStruct((2, _NP, _HID), _f32),
    mesh=_mesh,
    compiler_params=pltpu.CompilerParams(use_tc_tiling_on_sc=False),
    name="deg_scatter",
    scratch_types=[
        pltpu.VMEM((_NSUPP, _SCH), jnp.int32),
        pltpu.VMEM((_SCH, _HID), _f32),
        pltpu.VMEM_SHARED((_NP, _HID), _f32),
        pltpu.SemaphoreType.DMA,
    ],
)


def _make_combine(has_z, relu):
    def body(p_hbm, z_hbm, d_hbm, t_hbm, g_hbm,
             p0_v, p1_v, z_v, d_v, t_v, g_v):
        c = lax.axis_index("c")
        s = lax.axis_index("s")
        wid = c * 16 + s
        r0 = wid * _RPT
        pltpu.sync_copy(p_hbm.at[0, pl.ds(r0, _RPT)], p0_v)
        pltpu.sync_copy(p_hbm.at[1, pl.ds(r0, _RPT)], p1_v)
        if has_z:
            pltpu.sync_copy(z_hbm.at[pl.ds(r0, _RPT)], z_v)
        pltpu.sync_copy(d_hbm.at[pl.ds(r0, _RPT)], d_v)

        def row(r, carry):
            acc = (p0_v[r, :] + p1_v[r, :]) * d_v[r, :]
            if has_z:
                acc = acc + z_v[r, :]
            if relu:
                acc = jnp.maximum(acc, 0.0)
            t_v[r, :] = acc
            g_v[r, :] = acc * d_v[r, :]
            return carry

        lax.fori_loop(0, _RPT, row, 0)
        pltpu.sync_copy(t_v, t_hbm.at[pl.ds(r0, _RPT)])
        pltpu.sync_copy(g_v, g_hbm.at[pl.ds(r0, _RPT)])

    return pl.kernel(
        body,
        out_type=(jax.ShapeDtypeStruct((_NP, _HID), _f32),
                  jax.ShapeDtypeStruct((_NP, _HID), _f32)),
        mesh=_mesh,
        compiler_params=pltpu.CompilerParams(use_tc_tiling_on_sc=False),
        name="combine" + ("_z" if has_z else "") + ("_relu" if relu else ""),
        scratch_types=[
            pltpu.VMEM((_RPT, _HID), _f32),
            pltpu.VMEM((_RPT, _HID), _f32),
            pltpu.VMEM((_RPT, _HID), _f32),
            pltpu.VMEM((_RPT, _HID), _f32),
            pltpu.VMEM((_RPT, _HID), _f32),
            pltpu.VMEM((_RPT, _HID), _f32),
        ],
    )


_combine_z = _make_combine(True, False)
_combine_z_relu = _make_combine(True, True)
_combine_noz = _make_combine(False, False)


def _tc_mm_body(x_ref, w_ref, zs_ref):
    w = w_ref[...]
    wr = w.reshape(4, _DIN, _HID).transpose(1, 0, 2).reshape(_DIN, 4 * _HID)
    xz = jnp.dot(x_ref[...], wr, preferred_element_type=_f32)
    for k in range(4):
        zs_ref[k] = xz[:, _HID * k:_HID * (k + 1)]


_mm_call = pl.pallas_call(
    _tc_mm_body,
    out_shape=jax.ShapeDtypeStruct((4, _NP, _HID), _f32),
)


def _tc_dinv_body(degp_ref, z3_ref, g3_ref, d16_ref):
    deg = degp_ref[0] + degp_ref[1]
    dinv = jnp.where(deg > 0.0, lax.rsqrt(deg), 0.0)
    d16_ref[...] = dinv
    g3_ref[...] = dinv * z3_ref[...]


_dinv_call = pl.pallas_call(
    _tc_dinv_body,
    out_shape=(jax.ShapeDtypeStruct((_NP, _HID), _f32),
               jax.ShapeDtypeStruct((_NP, _HID), _f32)),
)


def _tc_final_body(u0, u1, u2, u3, w_ref, out_ref):
    h = jnp.concatenate([u0[...], u1[...], u2[...], u3[...]], axis=1)
    out_ref[...] = jnp.dot(h, w_ref[...], preferred_element_type=_f32)


_final_call = pl.pallas_call(
    _tc_final_body,
    out_shape=jax.ShapeDtypeStruct((_NP, _OUT), _f32),
)


def kernel(x, edge_index, W1, W2):
    src = edge_index[0].astype(jnp.int32)
    dst = edge_index[1].astype(jnp.int32)
    pad = jnp.full((_EP - _E,), _NP - 1, jnp.int32)
    dummy = jnp.full((_TILES, _NCHUNKP - _NCHUNK, _CHUNK), _NP - 1, jnp.int32)
    src3 = jnp.concatenate(
        [jnp.concatenate([src, pad]).reshape(_TILES, _NCHUNK, _CHUNK), dummy],
        axis=1).reshape(_TILES, _NSUPP, _SCH)
    dst3 = jnp.concatenate(
        [jnp.concatenate([dst, pad]).reshape(_TILES, _NCHUNK, _CHUNK), dummy],
        axis=1).reshape(_TILES, _NSUPP, _SCH)
    xp = jnp.zeros((_NP, _DIN), _f32).at[:_N].set(x)
    zer = jnp.zeros((_NP, _HID), _f32)
    ones = jnp.ones((_SCH, _HID), _f32)

    degp = _deg_call(ones, dst3, zer)
    zs = _mm_call(xp, W1)
    g, d16 = _dinv_call(degp, zs[3])

    p = _scatter_call(g, src3, dst3, zer)
    _, g = _combine_z(p, zs[2], d16)
    p = _scatter_call(g, src3, dst3, zer)
    _, g = _combine_z(p, zs[1], d16)
    p = _scatter_call(g, src3, dst3, zer)
    u0, g = _combine_z_relu(p, zs[0], d16)

    us = [u0]
    for _ in range(3):
        p = _scatter_call(g, src3, dst3, zer)
        u, g = _combine_noz(p, zer, d16)
        us.append(u)

    out = _final_call(us[0], us[1], us[2], us[3], W2)
    return out[:_N]


# speedup vs baseline: 3.5910x; 1.8615x over previous
import jax
import jax.numpy as jnp
from jax import lax
from jax.experimental import pallas as pl
from jax.experimental.pallas import tpu as pltpu
from jax.experimental.pallas import tpu_sc as plsc

_N = 10000
_NP = 10240
_E = 320000
_DIN = 128
_HID = 16
_OUT = 128

_TILES = 32
_CHUNK = 128
_SUP = 1
_NSUP = 80
_NSUPP = 82
_SCH = _SUP * _CHUNK
_NCHUNK = 80
_NCHUNKP = 82
_EPT = _CHUNK * _NCHUNK
_EP = _EPT * _TILES
_RPT = _NP // _TILES
_RPS = _NP // 16

_mesh = plsc.VectorSubcoreMesh(core_axis_name="c", subcore_axis_name="s")
_f32 = jnp.float32


def _hop_scatter_body(g_hbm, src_hbm, dst_hbm, zer_hbm, p_hbm,
                      src_v, dst_v, rb0, rb1, rb2, rb3, acc_sh,
                      sg0, sg1, sg2, sg3, ss0, ss1, ss2, ss3):
    c = lax.axis_index("c")
    s = lax.axis_index("s")
    wid = c * 16 + s
    pltpu.sync_copy(src_hbm.at[wid], src_v)
    pltpu.sync_copy(dst_hbm.at[wid], dst_v)
    r0 = s * _RPS
    pltpu.sync_copy(zer_hbm.at[pl.ds(r0, _RPS)], acc_sh.at[pl.ds(r0, _RPS)])
    plsc.subcore_barrier()

    rows = [rb0, rb1, rb2, rb3]
    sg = [sg0, sg1, sg2, sg3]
    ss = [ss0, ss1, ss2, ss3]

    def g_start(e, j):
        pltpu.async_copy(g_hbm.at[src_v.at[e]], rows[j], sg[j])

    def g_wait(e, j):
        pltpu.make_async_copy(g_hbm.at[src_v.at[e]], rows[j], sg[j]).wait()

    def s_start(e, j):
        pltpu.async_copy(rows[j], acc_sh.at[dst_v.at[e]], ss[j], add=True)

    def s_wait(e, j):
        pltpu.make_async_copy(rows[j], acc_sh.at[dst_v.at[e]], ss[j]).wait()

    g_start(0, 0)
    g_start(1, 1)
    g_wait(0, 0); s_start(0, 0); g_start(2, 2)
    g_wait(1, 1); s_start(1, 1); g_start(3, 3)
    g_wait(2, 2); s_start(2, 2); s_wait(0, 0); g_start(4, 0)
    g_wait(3, 3); s_start(3, 3); s_wait(1, 1); g_start(5, 1)

    def step(i, carry):
        e = 4 * i
        for j in range(4):
            g_wait(e + j, j)
            s_start(e + j, j)
            s_wait(e + j - 2, (j + 2) % 4)
            g_start(e + j + 2, (j + 2) % 4)
        return carry

    lax.fori_loop(1, _NSUP // 4, step, 0)
    s_wait(_NSUP - 2, 2)
    s_wait(_NSUP - 1, 3)
    g_wait(_NSUP, 0)
    g_wait(_NSUP + 1, 1)
    plsc.subcore_barrier()
    pltpu.sync_copy(acc_sh.at[pl.ds(r0, _RPS)], p_hbm.at[c, pl.ds(r0, _RPS)])


_scatter_call = pl.kernel(
    _hop_scatter_body,
    out_type=jax.ShapeDtypeStruct((2, _NP, _HID), _f32),
    mesh=_mesh,
    compiler_params=pltpu.CompilerParams(use_tc_tiling_on_sc=False),
    name="hop_scatter",
    scratch_types=[
        pltpu.VMEM((_NSUPP, _SCH), jnp.int32),
        pltpu.VMEM((_NSUPP, _SCH), jnp.int32),
        pltpu.VMEM((_SCH, _HID), _f32),
        pltpu.VMEM((_SCH, _HID), _f32),
        pltpu.VMEM((_SCH, _HID), _f32),
        pltpu.VMEM((_SCH, _HID), _f32),
        pltpu.VMEM_SHARED((_NP, _HID), _f32),
        pltpu.SemaphoreType.DMA,
        pltpu.SemaphoreType.DMA,
        pltpu.SemaphoreType.DMA,
        pltpu.SemaphoreType.DMA,
        pltpu.SemaphoreType.DMA,
        pltpu.SemaphoreType.DMA,
        pltpu.SemaphoreType.DMA,
        pltpu.SemaphoreType.DMA,
    ],
)


def _deg_scatter_body(ones_hbm, dst_hbm, zer_hbm, p_hbm,
                      dst_v, rows_v, acc_sh, sem):
    c = lax.axis_index("c")
    s = lax.axis_index("s")
    wid = c * 16 + s
    pltpu.sync_copy(dst_hbm.at[wid], dst_v)
    pltpu.sync_copy(ones_hbm, rows_v)
    r0 = s * _RPS
    pltpu.sync_copy(zer_hbm.at[pl.ds(r0, _RPS)], acc_sh.at[pl.ds(r0, _RPS)])
    plsc.subcore_barrier()

    def fire(j, carry):
        pltpu.async_copy(rows_v, acc_sh.at[dst_v.at[j]], sem, add=True)
        return carry

    def drain(j, carry):
        pltpu.make_async_copy(rows_v, acc_sh.at[dst_v.at[j]], sem).wait()
        return carry

    lax.fori_loop(0, _NSUP, fire, 0)
    lax.fori_loop(0, _NSUP, drain, 0)
    plsc.subcore_barrier()
    pltpu.sync_copy(acc_sh.at[pl.ds(r0, _RPS)], p_hbm.at[c, pl.ds(r0, _RPS)])


_deg_call = pl.kernel(
    _deg_scatter_body,
    out_type=jax.ShapeDtypeStruct((2, _NP, _HID), _f32),
    mesh=_mesh,
    compiler_params=pltpu.CompilerParams(use_tc_tiling_on_sc=False),
    name="deg_scatter",
    scratch_types=[
        pltpu.VMEM((_NSUPP, _SCH), jnp.int32),
        pltpu.VMEM((_SCH, _HID), _f32),
        pltpu.VMEM_SHARED((_NP, _HID), _f32),
        pltpu.SemaphoreType.DMA,
    ],
)


def _make_combine(has_z, relu):
    def body(p_hbm, z_hbm, d_hbm, t_hbm, g_hbm,
             p0_v, p1_v, z_v, d_v, t_v, g_v):
        c = lax.axis_index("c")
        s = lax.axis_index("s")
        wid = c * 16 + s
        r0 = wid * _RPT
        pltpu.sync_copy(p_hbm.at[0, pl.ds(r0, _RPT)], p0_v)
        pltpu.sync_copy(p_hbm.at[1, pl.ds(r0, _RPT)], p1_v)
        if has_z:
            pltpu.sync_copy(z_hbm.at[pl.ds(r0, _RPT)], z_v)
        pltpu.sync_copy(d_hbm.at[pl.ds(r0, _RPT)], d_v)

        def row(r, carry):
            acc = (p0_v[r, :] + p1_v[r, :]) * d_v[r, :]
            if has_z:
                acc = acc + z_v[r, :]
            if relu:
                acc = jnp.maximum(acc, 0.0)
            t_v[r, :] = acc
            g_v[r, :] = acc * d_v[r, :]
            return carry

        lax.fori_loop(0, _RPT, row, 0)
        pltpu.sync_copy(t_v, t_hbm.at[pl.ds(r0, _RPT)])
        pltpu.sync_copy(g_v, g_hbm.at[pl.ds(r0, _RPT)])

    return pl.kernel(
        body,
        out_type=(jax.ShapeDtypeStruct((_NP, _HID), _f32),
                  jax.ShapeDtypeStruct((_NP, _HID), _f32)),
        mesh=_mesh,
        compiler_params=pltpu.CompilerParams(use_tc_tiling_on_sc=False),
        name="combine" + ("_z" if has_z else "") + ("_relu" if relu else ""),
        scratch_types=[
            pltpu.VMEM((_RPT, _HID), _f32),
            pltpu.VMEM((_RPT, _HID), _f32),
            pltpu.VMEM((_RPT, _HID), _f32),
            pltpu.VMEM((_RPT, _HID), _f32),
            pltpu.VMEM((_RPT, _HID), _f32),
            pltpu.VMEM((_RPT, _HID), _f32),
        ],
    )


_combine_z = _make_combine(True, False)
_combine_z_relu = _make_combine(True, True)
_combine_noz = _make_combine(False, False)


def _tc_mm_body(x_ref, w_ref, zs_ref):
    w = w_ref[...]
    wr = w.reshape(4, _DIN, _HID).transpose(1, 0, 2).reshape(_DIN, 4 * _HID)
    xz = jnp.dot(x_ref[...], wr, preferred_element_type=_f32)
    for k in range(4):
        zs_ref[k] = xz[:, _HID * k:_HID * (k + 1)]


_mm_call = pl.pallas_call(
    _tc_mm_body,
    out_shape=jax.ShapeDtypeStruct((4, _NP, _HID), _f32),
)


def _tc_dinv_body(degp_ref, z3_ref, g3_ref, d16_ref):
    deg = degp_ref[0] + degp_ref[1]
    dinv = jnp.where(deg > 0.0, lax.rsqrt(deg), 0.0)
    d16_ref[...] = dinv
    g3_ref[...] = dinv * z3_ref[...]


_dinv_call = pl.pallas_call(
    _tc_dinv_body,
    out_shape=(jax.ShapeDtypeStruct((_NP, _HID), _f32),
               jax.ShapeDtypeStruct((_NP, _HID), _f32)),
)


def _tc_final_body(u0, u1, u2, u3, w_ref, out_ref):
    h = jnp.concatenate([u0[...], u1[...], u2[...], u3[...]], axis=1)
    out_ref[...] = jnp.dot(h, w_ref[...], preferred_element_type=_f32)


_final_call = pl.pallas_call(
    _tc_final_body,
    out_shape=jax.ShapeDtypeStruct((_NP, _OUT), _f32),
)


def kernel(x, edge_index, W1, W2):
    src = edge_index[0].astype(jnp.int32)
    dst = edge_index[1].astype(jnp.int32)
    pad = _N + jnp.arange(_EP - _E, dtype=jnp.int32) % (_NP - _N)
    dummy = _N + (jnp.arange(_TILES * (_NCHUNKP - _NCHUNK) * _CHUNK,
                             dtype=jnp.int32) % (_NP - _N)
                  ).reshape(_TILES, _NCHUNKP - _NCHUNK, _CHUNK)
    src3 = jnp.concatenate(
        [jnp.concatenate([src, pad]).reshape(_TILES, _NCHUNK, _CHUNK), dummy],
        axis=1).reshape(_TILES, _NSUPP, _SCH)
    dst3 = jnp.concatenate(
        [jnp.concatenate([dst, pad]).reshape(_TILES, _NCHUNK, _CHUNK), dummy],
        axis=1).reshape(_TILES, _NSUPP, _SCH)
    xp = jnp.zeros((_NP, _DIN), _f32).at[:_N].set(x)
    zer = jnp.zeros((_NP, _HID), _f32)
    ones = jnp.ones((_SCH, _HID), _f32)

    degp = _deg_call(ones, dst3, zer)
    zs = _mm_call(xp, W1)
    g, d16 = _dinv_call(degp, zs[3])

    p = _scatter_call(g, src3, dst3, zer)
    _, g = _combine_z(p, zs[2], d16)
    p = _scatter_call(g, src3, dst3, zer)
    _, g = _combine_z(p, zs[1], d16)
    p = _scatter_call(g, src3, dst3, zer)
    u0, g = _combine_z_relu(p, zs[0], d16)

    us = [u0]
    for _ in range(3):
        p = _scatter_call(g, src3, dst3, zer)
        u, g = _combine_noz(p, zer, d16)
        us.append(u)

    out = _final_call(us[0], us[1], us[2], us[3], W2)
    return out[:_N]
